# baseline (speedup 1.0000x reference)
import jax
import jax.numpy as jnp
from jax.experimental import pallas as pl
from jax.experimental.pallas import tpu as pltpu
from jax.experimental.pallas import tpu_sc as plsc

B = 32
NPG = 100
N = B * NPG
DEG = 16
H = 4
HF = 512
HHF = H * HF
MPAD = 112


def _mbuild(src_flat):
    mesh = plsc.VectorSubcoreMesh(core_axis_name="c", subcore_axis_name="s",
                                  num_cores=2, num_subcores=16)

    @pl.kernel(
        out_type=jax.ShapeDtypeStruct((B, NPG, MPAD), jnp.float32),
        mesh=mesh,
        scratch_types=[pltpu.VMEM((NPG * DEG,), jnp.int32),
                       pltpu.VMEM((NPG, MPAD), jnp.float32),
                       pltpu.SemaphoreType.DMA],
        compiler_params=pltpu.CompilerParams(needs_layout_passes=False))
    def mk(src_hbm, o_hbm, src_vmem, m_vmem, sem):
        g = jax.lax.axis_index("c") * 16 + jax.lax.axis_index("s")
        pltpu.async_copy(
            src_hbm.at[pl.ds(g * NPG * DEG, NPG * DEG)], src_vmem, sem
        ).wait()
        iota = jax.lax.iota(jnp.int32, 16)
        zeros = jnp.zeros((16,), jnp.float32)
        ones = jnp.ones((16,), jnp.float32)

        @pl.loop(0, NPG)
        def _(d):
            @pl.loop(0, MPAD // 16)
            def _(c):
                m_vmem[d, pl.ds(c * 16, 16)] = zeros

        @pl.loop(0, (NPG + 15) // 16)
        def _(grp):
            rows = iota + grp * 16
            mask = rows < NPG
            rows_c = jnp.minimum(rows, NPG - 1)

            @pl.loop(0, DEG)
            def _(k):
                srcs = plsc.load_gather(src_vmem, [rows_c * DEG + k])
                plsc.addupdate_scatter(m_vmem, [rows_c, srcs - g * NPG],
                                       ones, mask=mask)

        pltpu.async_copy(m_vmem, o_hbm.at[g], sem).wait()

    return mk(src_flat)


def _elu(x):
    return jnp.where(x > 0, x, jnp.exp(x) - 1.0)


def _alr_pack(al, ar):
    rows = jnp.arange(HHF)[:, None] // HF
    cols = jnp.arange(2 * H)[None, :]
    alf = al.reshape(-1, 1)
    arf = ar.reshape(-1, 1)
    return (jnp.where(cols == rows, alf, 0.0)
            + jnp.where(cols - H == rows, arf, 0.0))


def _attention(feat, elr, m):
    edge = m > 0.0
    elt = jnp.swapaxes(elr[:, :H], 0, 1)
    outs = []
    for h in range(H):
        f = elt[h:h + 1, :] + elr[:, H + h:H + h + 1]
        f = jnp.where(f >= 0, f, 0.2 * f)
        fmax = jnp.max(jnp.where(edge, f, -1e30), axis=1, keepdims=True)
        ex = m * jnp.exp(f - fmax)
        a = ex / jnp.sum(ex, axis=1, keepdims=True)
        outs.append(jnp.dot(a, feat[:, h * HF:(h + 1) * HF],
                            preferred_element_type=jnp.float32))
    return jnp.concatenate(outs, axis=1)


G = 4


def _node_kernel(x_ref, m_ref, w1_ref, alr1_ref, b1_ref,
                 w2_ref, alr2_ref, b2_ref,
                 wfc_ref, bfc_ref, wh_ref, bh_ref, o_ref):
    for g in range(G):
        x = x_ref[g]
        m = m_ref[g][:, :NPG]
        feat1 = jnp.dot(x, w1_ref[...], preferred_element_type=jnp.float32)
        elr1 = jnp.dot(feat1, alr1_ref[...],
                       preferred_element_type=jnp.float32)
        h1 = _attention(feat1, elr1, m) + b1_ref[...]
        feat2 = jnp.dot(h1, w2_ref[...], preferred_element_type=jnp.float32)
        elr2 = jnp.dot(feat2, alr2_ref[...],
                       preferred_element_type=jnp.float32)
        h2 = _attention(feat2, elr2, m) + b2_ref[...]
        y = _elu(jnp.dot(h2, wfc_ref[...],
                         preferred_element_type=jnp.float32) + bfc_ref[...])
        for i in range(8):
            y = _elu(jnp.dot(y, wh_ref[i],
                             preferred_element_type=jnp.float32)
                     + bh_ref[i][None, :])
        o_ref[g] = y


def _node(x3, m3, p, alr1, alr2):
    k = x3.shape[2]
    return pl.pallas_call(
        _node_kernel,
        grid=(B // G,),
        in_specs=[
            pl.BlockSpec((G, NPG, k), lambda i: (i, 0, 0)),
            pl.BlockSpec((G, NPG, MPAD), lambda i: (i, 0, 0)),
            pl.BlockSpec(p['W_gat1'].shape, lambda i: (0, 0)),
            pl.BlockSpec(alr1.shape, lambda i: (0, 0)),
            pl.BlockSpec((1, HHF), lambda i: (0, 0)),
            pl.BlockSpec(p['W_gat2'].shape, lambda i: (0, 0)),
            pl.BlockSpec(alr2.shape, lambda i: (0, 0)),
            pl.BlockSpec((1, HHF), lambda i: (0, 0)),
            pl.BlockSpec(p['W_fc'].shape, lambda i: (0, 0)),
            pl.BlockSpec((1, HF), lambda i: (0, 0)),
            pl.BlockSpec(p['W_hid'].shape, lambda i: (0, 0, 0)),
            pl.BlockSpec(p['b_hid'].shape, lambda i: (0, 0)),
        ],
        out_specs=pl.BlockSpec((G, NPG, HF), lambda i: (i, 0, 0)),
        out_shape=jax.ShapeDtypeStruct((B, NPG, HF), jnp.float32),
        compiler_params=pltpu.CompilerParams(
            dimension_semantics=("parallel",)),
    )(x3, m3, p['W_gat1'], alr1, p['b_gat1'].reshape(1, -1),
      p['W_gat2'], alr2, p['b_gat2'].reshape(1, -1),
      p['W_fc'], p['b_fc'].reshape(1, -1), p['W_hid'], p['b_hid'])


def _bn_cols(x):
    m = jnp.mean(x, axis=0, keepdims=True)
    v = jnp.mean((x - m) * (x - m), axis=0, keepdims=True)
    return (x - m) / jnp.sqrt(v + 1e-5)


def _head_kernel(img_ref, txt_ref, h_ref, pos_ref,
                 gswin_ref, bswin_ref, wswin_ref, bbswin_ref,
                 gtext_ref, btext_ref, wtext_ref, bbtext_ref,
                 ggat_ref, bgat_ref, wfcgat_ref, bfcgat_ref,
                 gbbox_ref, bbbox_ref, wfcbbox_ref, bfcbbox_ref,
                 gfx_ref, bfx_ref, wfx_ref,
                 gfh_ref, bfh_ref, wfh_ref,
                 gfp_ref, bfp_ref, wfp_ref,
                 gft_ref, bft_ref, wft_ref, bfin_ref,
                 o_ref):
    x = _bn_cols(img_ref[...]) * gswin_ref[...] + bswin_ref[...]
    x = _elu(jnp.dot(x, wswin_ref[...],
                     preferred_element_type=jnp.float32) + bbswin_ref[...])
    t = _bn_cols(txt_ref[...]) * gtext_ref[...] + btext_ref[...]
    t = _elu(jnp.dot(t, wtext_ref[...],
                     preferred_element_type=jnp.float32) + bbtext_ref[...])

    h = h_ref[...]
    m = jnp.mean(h, axis=(0, 2), keepdims=True)
    v = jnp.mean((h - m) * (h - m), axis=(0, 2), keepdims=True)
    hn = (h - m) / jnp.sqrt(v + 1e-5) * ggat_ref[...] + bgat_ref[...]
    hg = _elu(jnp.dot(hn.reshape(N, HF), wfcgat_ref[...],
                      preferred_element_type=jnp.float32) + bfcgat_ref[...])
    mh = jnp.mean(hg.reshape(B, NPG, 480), axis=1)

    pos = pos_ref[...]
    pm = jnp.mean(pos, axis=(0, 2), keepdims=True)
    pv = jnp.mean((pos - pm) * (pos - pm), axis=(0, 2), keepdims=True)
    pn = (pos - pm) / jnp.sqrt(pv + 1e-5) * gbbox_ref[...] + bbbox_ref[...]
    pg = _elu(jnp.dot(pn.reshape(N, 4), wfcbbox_ref[...],
                      preferred_element_type=jnp.float32) + bfcbbox_ref[...])
    mp = jnp.mean(pg.reshape(B, NPG, 32), axis=1)

    out = jnp.dot(_bn_cols(x) * gfx_ref[...] + bfx_ref[...], wfx_ref[...],
                  preferred_element_type=jnp.float32)
    out = out + jnp.dot(_bn_cols(mh) * gfh_ref[...] + bfh_ref[...],
                        wfh_ref[...], preferred_element_type=jnp.float32)
    out = out + jnp.dot(_bn_cols(mp) * gfp_ref[...] + bfp_ref[...],
                        wfp_ref[...], preferred_element_type=jnp.float32)
    out = out + jnp.dot(_bn_cols(t) * gft_ref[...] + bft_ref[...],
                        wft_ref[...], preferred_element_type=jnp.float32)
    o_ref[...] = out + bfin_ref[...]


def _full(x):
    return pl.BlockSpec(x.shape, lambda: tuple(0 for _ in x.shape))


def _head(img, txt, h3, pos3, p):
    gfx, gfh, gfp, gft = (p['g_final'][:512].reshape(1, -1),
                          p['g_final'][512:992].reshape(1, -1),
                          p['g_final'][992:1024].reshape(1, -1),
                          p['g_final'][1024:].reshape(1, -1))
    bfx, bfh, bfp, bft = (p['b_final_bn'][:512].reshape(1, -1),
                          p['b_final_bn'][512:992].reshape(1, -1),
                          p['b_final_bn'][992:1024].reshape(1, -1),
                          p['b_final_bn'][1024:].reshape(1, -1))
    wfx, wfh, wfp, wft = (p['W_final'][:512], p['W_final'][512:992],
                          p['W_final'][992:1024], p['W_final'][1024:])
    args = (img, txt, h3, pos3,
            p['g_swin'].reshape(1, -1), p['b_swin'].reshape(1, -1),
            p['W_swin'], p['bb_swin'].reshape(1, -1),
            p['g_text'].reshape(1, -1), p['b_text'].reshape(1, -1),
            p['W_text'], p['bb_text'].reshape(1, -1),
            p['g_gat'].reshape(1, NPG, 1), p['b_gat'].reshape(1, NPG, 1),
            p['W_fcgat'], p['b_fcgat'].reshape(1, -1),
            p['g_bbox'].reshape(1, NPG, 1), p['b_bbox'].reshape(1, NPG, 1),
            p['W_fcbbox'], p['b_fcbbox'].reshape(1, -1),
            gfx, bfx, wfx, gfh, bfh, wfh, gfp, bfp, wfp, gft, bft, wft,
            p['b_final'].reshape(1, -1))
    return pl.pallas_call(
        _head_kernel,
        in_specs=[_full(a) for a in args],
        out_specs=pl.BlockSpec((B, 6), lambda: (0, 0)),
        out_shape=jax.ShapeDtypeStruct((B, 6), jnp.float32),
    )(*args)


def kernel(img_embedding, func_text_embedding, unix_emb, func_emb, pos_emb,
           params, edge_src, edge_dst):
    p = params
    alr1 = _alr_pack(p['al1'], p['ar1'])
    alr2 = _alr_pack(p['al2'], p['ar2'])

    m3 = _mbuild(edge_src.astype(jnp.int32))
    hm = _node(unix_emb.reshape(B, NPG, -1), m3, p, alr1, alr2)
    return _head(img_embedding, func_text_embedding,
                 hm, pos_emb.reshape(B, NPG, 4), p)
